# Initial kernel scaffold; baseline (speedup 1.0000x reference)
import jax
import jax.numpy as jnp
from jax.experimental import pallas as pl


def kernel(feats, ancs, ancs_valid, W_b, b_b, W_cls, b_cls, W_reg, b_reg):
    raise NotImplementedError("write your pallas kernel here")


# speedup vs baseline: 101.5208x; 101.5208x over previous
import functools

import jax
import jax.numpy as jnp
from jax.experimental import pallas as pl
from jax.experimental.pallas import tpu as pltpu

GH, GW, NA, CIN, CMID = 32, 32, 9, 768, 256
NPOS = GH * GW
N = NPOS * NA
BK = 128
NB = N // BK
MAX_POST = 2000
NOUT = 2048
NOB = NOUT // BK
IOU_T = 0.7


def _heads_kernel(x_ref, wb_ref, bb_ref, wc_ref, bc_ref, wr_ref, br_ref,
                  valid_ref, cls_ref, reg_ref, sm_ref):
    h = jnp.maximum(
        jnp.dot(x_ref[:], wb_ref[:], preferred_element_type=jnp.float32)
        + bb_ref[:], 0.0)
    logits = jnp.dot(h, wc_ref[:], preferred_element_type=jnp.float32) + bc_ref[:]
    cls = jax.nn.sigmoid(logits)
    reg = jnp.dot(h, wr_ref[:], preferred_element_type=jnp.float32) + br_ref[:]
    cls_ref[:] = cls
    reg_ref[:] = reg
    sm_ref[:] = jnp.where(valid_ref[:] > 0.0, cls, -jnp.inf)


def _nms_kernel(s_row_ref, s_col_ref, a4_ref, out_ref,
                rank_c, sP, keep_r, sup_r, destk_c):
    f32 = jnp.float32
    a0 = a4_ref[0:1, :]
    a1 = a4_ref[1:2, :]
    a2 = a4_ref[2:3, :]
    a3 = a4_ref[3:4, :]
    c0 = a0 + a0 * a2
    c1 = a1 + a1 * a3
    w0 = a2 * jnp.exp(a2)
    w1 = a3 * jnp.exp(a3)
    P0 = c0 - w0 * 0.5
    P1 = c1 - w1 * 0.5
    P2 = c0 + w0 * 0.5
    P3 = c1 + w1 * 0.5
    P4 = jnp.concatenate([P0, P1, P2, P3], axis=0)

    idx_row = jax.lax.broadcasted_iota(jnp.int32, (1, N), 1)
    iota_c = jax.lax.broadcasted_iota(jnp.int32, (BK, 1), 0)
    iota_r = jax.lax.broadcasted_iota(jnp.int32, (1, BK), 1)
    s_row = s_row_ref[:]

    def rank_body(bi, carry):
        base = bi * BK
        sc = s_col_ref[pl.ds(base, BK), :]
        ic = iota_c + base
        gt = (s_row > sc).astype(f32)
        tie = ((s_row == sc) & (idx_row > ic)).astype(f32)
        rank_c[pl.ds(base, BK), :] = jnp.sum(gt + tie, axis=1, keepdims=True)
        return carry

    jax.lax.fori_loop(0, NB, rank_body, 0, unroll=False)

    rank_all = rank_c[:]

    def sort_body(db, carry):
        q = (iota_r + db * BK).astype(f32)
        oh = (rank_all == q).astype(f32)
        blk = jnp.dot(P4, oh, preferred_element_type=f32,
                      precision=jax.lax.Precision.HIGHEST)
        sP[:, pl.ds(db * BK, BK)] = blk
        return carry

    jax.lax.fori_loop(0, NB, sort_body, 0, unroll=False)

    sy1 = sP[0:1, :]
    sx1 = sP[1:2, :]
    sy2 = sP[2:3, :]
    sx2 = sP[3:4, :]
    sarea = (sy2 - sy1) * (sx2 - sx1)
    sup_r[:] = jnp.zeros((1, N), f32)
    strict_upper = (iota_c < iota_r).astype(f32)

    def nms_body(b, carry):
        base = b * BK
        y1r = sP[0:1, pl.ds(base, BK)]
        x1r = sP[1:2, pl.ds(base, BK)]
        y2r = sP[2:3, pl.ds(base, BK)]
        x2r = sP[3:4, pl.ds(base, BK)]
        ar = (y2r - y1r) * (x2r - x1r)
        y1c = jnp.transpose(y1r, (1, 0))
        x1c = jnp.transpose(x1r, (1, 0))
        y2c = jnp.transpose(y2r, (1, 0))
        x2c = jnp.transpose(x2r, (1, 0))
        ac = jnp.transpose(ar, (1, 0))
        supc = jnp.transpose(sup_r[0:1, pl.ds(base, BK)], (1, 0))
        ih = jnp.minimum(y2c, y2r) - jnp.maximum(y1c, y1r)
        iw = jnp.minimum(x2c, x2r) - jnp.maximum(x1c, x1r)
        inter = jnp.maximum(ih, 0.0) * jnp.maximum(iw, 0.0)
        Sb = (inter > IOU_T * (ac + ar - inter + 1e-9)).astype(f32) * strict_upper
        init_r = jnp.transpose((supc == 0.0).astype(f32), (1, 0))

        def fcond(st):
            it, changed, _ = st
            return (it < BK + 2) & changed

        def fbody(st):
            it, _, k = st
            su = jnp.dot(k, Sb, preferred_element_type=f32)
            nk = init_r * (su == 0.0).astype(f32)
            return it + 1, jnp.any(nk != k), nk

        _, _, k_row = jax.lax.while_loop(fcond, fbody, (0, True, init_r))
        keep_r[0:1, pl.ds(base, BK)] = k_row
        k_col = jnp.transpose(k_row, (1, 0))
        IH = jnp.minimum(y2c, sy2) - jnp.maximum(y1c, sy1)
        IW = jnp.minimum(x2c, sx2) - jnp.maximum(x1c, sx1)
        INTER = jnp.maximum(IH, 0.0) * jnp.maximum(IW, 0.0)
        SUP = (INTER > IOU_T * (ac + sarea - INTER + 1e-9)).astype(f32) * k_col
        newsup = (jnp.sum(SUP, axis=0, keepdims=True) > 0.0).astype(f32)
        sup_r[:] = jnp.maximum(sup_r[:], newsup)
        return carry

    jax.lax.fori_loop(0, NB, nms_body, 0, unroll=False)

    def dest_body(b, nkept):
        krb = keep_r[0:1, pl.ds(b * BK, BK)]
        excl = jnp.dot(krb, strict_upper, preferred_element_type=f32)
        destk = jnp.where(krb > 0.0, excl + nkept, -1.0)
        destk_c[pl.ds(b * BK, BK), :] = jnp.transpose(destk, (1, 0))
        return nkept + jnp.sum(krb)

    nkept = jax.lax.fori_loop(0, NB, dest_body, 0.0, unroll=False)

    destk_all = destk_c[:]
    sb0 = sP[:, 0:1]

    def out_body(ob, carry):
        p = (iota_r + ob * BK).astype(f32)
        oh = (destk_all == p).astype(f32)
        blk = jnp.dot(sP[:, :], oh, preferred_element_type=f32,
                      precision=jax.lax.Precision.HIGHEST)
        blk = blk + (p >= nkept).astype(f32) * sb0
        out_ref[:, pl.ds(ob * BK, BK)] = blk
        return carry

    jax.lax.fori_loop(0, NOB, out_body, 0, unroll=False)


@functools.partial(jax.jit, static_argnames=())
def _run(feats, ancs, ancs_valid, W_b, b_b, W_cls, b_cls, W_reg, b_reg):
    x = feats.reshape(NPOS, CIN)
    valid2d = ancs_valid.reshape(NPOS, NA)
    cls2d, reg2d, sm2d = pl.pallas_call(
        _heads_kernel,
        out_shape=(
            jax.ShapeDtypeStruct((NPOS, NA), jnp.float32),
            jax.ShapeDtypeStruct((NPOS, NA * 4), jnp.float32),
            jax.ShapeDtypeStruct((NPOS, NA), jnp.float32),
        ),
    )(x, W_b, b_b.reshape(1, CMID), W_cls, b_cls.reshape(1, NA),
      W_reg, b_reg.reshape(1, NA * 4), valid2d)

    s_row = sm2d.reshape(1, N)
    s_col = sm2d.reshape(N, 1)
    a4 = ancs.reshape(N, 4).T

    out4 = pl.pallas_call(
        _nms_kernel,
        out_shape=jax.ShapeDtypeStruct((4, NOUT), jnp.float32),
        scratch_shapes=[
            pltpu.VMEM((N, 1), jnp.float32),
            pltpu.VMEM((4, N), jnp.float32),
            pltpu.VMEM((1, N), jnp.float32),
            pltpu.VMEM((1, N), jnp.float32),
            pltpu.VMEM((N, 1), jnp.float32),
        ],
    )(s_row, s_col, a4)

    cls_pred = cls2d.reshape(1, GH, GW, NA)
    reg_pred = reg2d.reshape(1, GH, GW, NA, 4)
    boxes = out4.T[:MAX_POST, :]
    return (cls_pred, reg_pred, boxes)


def kernel(feats, ancs, ancs_valid, W_b, b_b, W_cls, b_cls, W_reg, b_reg):
    return _run(feats, ancs, ancs_valid, W_b, b_b, W_cls, b_cls, W_reg, b_reg)


# speedup vs baseline: 107.3994x; 1.0579x over previous
import functools

import jax
import jax.numpy as jnp
from jax.experimental import pallas as pl
from jax.experimental.pallas import tpu as pltpu

GH, GW, NA, CIN, CMID = 32, 32, 9, 768, 256
NPOS = GH * GW
N = NPOS * NA
BK = 128
NB = N // BK
MAX_POST = 2000
NOUT = 2048
NOB = NOUT // BK
IOU_T = 0.7


def _heads_kernel(x_ref, wb_ref, bb_ref, wc_ref, bc_ref, wr_ref, br_ref,
                  valid_ref, cls_ref, reg_ref, sm_ref):
    h = jnp.maximum(
        jnp.dot(x_ref[:], wb_ref[:], preferred_element_type=jnp.float32)
        + bb_ref[:], 0.0)
    logits = jnp.dot(h, wc_ref[:], preferred_element_type=jnp.float32) + bc_ref[:]
    cls = jax.nn.sigmoid(logits)
    reg = jnp.dot(h, wr_ref[:], preferred_element_type=jnp.float32) + br_ref[:]
    cls_ref[:] = cls
    reg_ref[:] = reg
    sm_ref[:] = jnp.where(valid_ref[:] > 0.0, cls, -jnp.inf)


def _nms_kernel(s_row_ref, s_col_ref, a4_ref, out_ref,
                rank_c, sP, keep_r, sup_r, destk_c):
    f32 = jnp.float32
    a0 = a4_ref[0:1, :]
    a1 = a4_ref[1:2, :]
    a2 = a4_ref[2:3, :]
    a3 = a4_ref[3:4, :]
    c0 = a0 + a0 * a2
    c1 = a1 + a1 * a3
    w0 = a2 * jnp.exp(a2)
    w1 = a3 * jnp.exp(a3)
    P0 = c0 - w0 * 0.5
    P1 = c1 - w1 * 0.5
    P2 = c0 + w0 * 0.5
    P3 = c1 + w1 * 0.5
    P4 = jnp.concatenate([P0, P1, P2, P3], axis=0)

    idx_row = jax.lax.broadcasted_iota(jnp.int32, (1, N), 1)
    iota_c = jax.lax.broadcasted_iota(jnp.int32, (BK, 1), 0)
    iota_r = jax.lax.broadcasted_iota(jnp.int32, (1, BK), 1)
    s_row = s_row_ref[:]

    def rank_body(bi, carry):
        base = bi * BK
        sc = s_col_ref[pl.ds(base, BK), :]
        ic = iota_c + base
        gt = (s_row > sc).astype(f32)
        tie = ((s_row == sc) & (idx_row > ic)).astype(f32)
        rank_c[pl.ds(base, BK), :] = jnp.sum(gt + tie, axis=1, keepdims=True)
        return carry

    jax.lax.fori_loop(0, NB, rank_body, 0, unroll=False)

    rank_all = rank_c[:]

    def sort_body(db, carry):
        q = (iota_r + db * BK).astype(f32)
        oh = (rank_all == q).astype(f32)
        blk = jnp.dot(P4, oh, preferred_element_type=f32,
                      precision=jax.lax.Precision.HIGHEST)
        sP[:, pl.ds(db * BK, BK)] = blk
        return carry

    jax.lax.fori_loop(0, NB, sort_body, 0, unroll=False)

    sup_r[:] = jnp.zeros((1, N), f32)
    strict_upper = (iota_c < iota_r).astype(f32)

    def nms_body(b, carry):
        base = b * BK
        y1r = sP[0:1, pl.ds(base, BK)]
        x1r = sP[1:2, pl.ds(base, BK)]
        y2r = sP[2:3, pl.ds(base, BK)]
        x2r = sP[3:4, pl.ds(base, BK)]
        ar = (y2r - y1r) * (x2r - x1r)
        y1c = jnp.transpose(y1r, (1, 0))
        x1c = jnp.transpose(x1r, (1, 0))
        y2c = jnp.transpose(y2r, (1, 0))
        x2c = jnp.transpose(x2r, (1, 0))
        ac = jnp.transpose(ar, (1, 0))
        supc = jnp.transpose(sup_r[0:1, pl.ds(base, BK)], (1, 0))
        ih = jnp.minimum(y2c, y2r) - jnp.maximum(y1c, y1r)
        iw = jnp.minimum(x2c, x2r) - jnp.maximum(x1c, x1r)
        inter = jnp.maximum(ih, 0.0) * jnp.maximum(iw, 0.0)
        Sb = (inter > IOU_T * (ac + ar - inter + 1e-9)).astype(f32) * strict_upper
        init_r = jnp.transpose((supc == 0.0).astype(f32), (1, 0))

        def fcond(st):
            it, changed, _ = st
            return (it < BK + 2) & changed

        def fbody(st):
            it, _, k = st
            su = jnp.dot(k, Sb, preferred_element_type=f32)
            nk = init_r * (su == 0.0).astype(f32)
            return it + 1, jnp.any(nk != k), nk

        _, _, k_row = jax.lax.while_loop(fcond, fbody, (0, True, init_r))
        keep_r[0:1, pl.ds(base, BK)] = k_row
        k_col = jnp.transpose(k_row, (1, 0))
        tac = ac + 1e-9

        def chunk_body(bj, c):
            cb = bj * BK
            jy1 = sP[0:1, pl.ds(cb, BK)]
            jx1 = sP[1:2, pl.ds(cb, BK)]
            jy2 = sP[2:3, pl.ds(cb, BK)]
            jx2 = sP[3:4, pl.ds(cb, BK)]
            jar = (jy2 - jy1) * (jx2 - jx1)
            cih = jnp.minimum(y2c, jy2) - jnp.maximum(y1c, jy1)
            ciw = jnp.minimum(x2c, jx2) - jnp.maximum(x1c, jx1)
            cin_ = jnp.maximum(cih, 0.0) * jnp.maximum(ciw, 0.0)
            csup = (cin_ > IOU_T * (tac + jar - cin_)).astype(f32) * k_col
            news = (jnp.sum(csup, axis=0, keepdims=True) > 0.0).astype(f32)
            sup_r[0:1, pl.ds(cb, BK)] = jnp.maximum(sup_r[0:1, pl.ds(cb, BK)], news)
            return c

        jax.lax.fori_loop(b + 1, NB, chunk_body, 0, unroll=False)
        return carry

    jax.lax.fori_loop(0, NB, nms_body, 0, unroll=False)

    def dest_body(b, nkept):
        krb = keep_r[0:1, pl.ds(b * BK, BK)]
        excl = jnp.dot(krb, strict_upper, preferred_element_type=f32)
        destk = jnp.where(krb > 0.0, excl + nkept, -1.0)
        destk_c[pl.ds(b * BK, BK), :] = jnp.transpose(destk, (1, 0))
        return nkept + jnp.sum(krb)

    nkept = jax.lax.fori_loop(0, NB, dest_body, 0.0, unroll=False)

    destk_all = destk_c[:]
    sb0 = sP[:, 0:1]

    def out_body(ob, carry):
        p = (iota_r + ob * BK).astype(f32)
        oh = (destk_all == p).astype(f32)
        blk = jnp.dot(sP[:, :], oh, preferred_element_type=f32,
                      precision=jax.lax.Precision.HIGHEST)
        blk = blk + (p >= nkept).astype(f32) * sb0
        out_ref[:, pl.ds(ob * BK, BK)] = blk
        return carry

    jax.lax.fori_loop(0, NOB, out_body, 0, unroll=False)


@functools.partial(jax.jit, static_argnames=())
def _run(feats, ancs, ancs_valid, W_b, b_b, W_cls, b_cls, W_reg, b_reg):
    x = feats.reshape(NPOS, CIN)
    valid2d = ancs_valid.reshape(NPOS, NA)
    cls2d, reg2d, sm2d = pl.pallas_call(
        _heads_kernel,
        out_shape=(
            jax.ShapeDtypeStruct((NPOS, NA), jnp.float32),
            jax.ShapeDtypeStruct((NPOS, NA * 4), jnp.float32),
            jax.ShapeDtypeStruct((NPOS, NA), jnp.float32),
        ),
    )(x, W_b, b_b.reshape(1, CMID), W_cls, b_cls.reshape(1, NA),
      W_reg, b_reg.reshape(1, NA * 4), valid2d)

    s_row = sm2d.reshape(1, N)
    s_col = sm2d.reshape(N, 1)
    a4 = ancs.reshape(N, 4).T

    out4 = pl.pallas_call(
        _nms_kernel,
        out_shape=jax.ShapeDtypeStruct((4, NOUT), jnp.float32),
        scratch_shapes=[
            pltpu.VMEM((N, 1), jnp.float32),
            pltpu.VMEM((4, N), jnp.float32),
            pltpu.VMEM((1, N), jnp.float32),
            pltpu.VMEM((1, N), jnp.float32),
            pltpu.VMEM((N, 1), jnp.float32),
        ],
    )(s_row, s_col, a4)

    cls_pred = cls2d.reshape(1, GH, GW, NA)
    reg_pred = reg2d.reshape(1, GH, GW, NA, 4)
    boxes = out4.T[:MAX_POST, :]
    return (cls_pred, reg_pred, boxes)


def kernel(feats, ancs, ancs_valid, W_b, b_b, W_cls, b_cls, W_reg, b_reg):
    return _run(feats, ancs, ancs_valid, W_b, b_b, W_cls, b_cls, W_reg, b_reg)
